# baseline (speedup 1.0000x reference)
import functools

import jax
import jax.numpy as jnp
import numpy as np
from jax import lax
from jax.experimental import pallas as pl
from jax.experimental.pallas import tpu as pltpu
from jax.experimental.pallas import tpu_sc as plsc

_N = 10000
_D = 128
_NEI = 2
_S = 5
_NP = 10240
_QB = 256


def _flash_body(q_ref, k_ref, v_ref, ow_ref, ob_ref, o_ref):
    q = q_ref[0]
    k = k_ref[0]
    v = v_ref[0]
    s = jax.lax.dot_general(q, k, (((1,), (1,)), ((), ())),
                            preferred_element_type=jnp.float32)
    s = s * (1.0 / np.sqrt(_D).astype(np.float32))
    kidx = jax.lax.broadcasted_iota(jnp.int32, s.shape, 1)
    s = jnp.where(kidx < _N, s, -1e30)
    m = jnp.max(s, axis=-1, keepdims=True)
    p = jnp.exp(s - m)
    l = jnp.sum(p, axis=-1, keepdims=True)
    o = jax.lax.dot_general(p.astype(jnp.bfloat16), v, (((1,), (0,)), ((), ())),
                            preferred_element_type=jnp.float32) / l
    o = jax.lax.dot_general(o, ow_ref[0], (((1,), (1,)), ((), ())),
                            preferred_element_type=jnp.float32)
    o_ref[0] = o + ob_ref[0]


def _flash_attention(q, k, v, out_w, out_b):
    grid = (_NEI, _NP // _QB)
    q = q.astype(jnp.bfloat16)
    k = k.astype(jnp.bfloat16)
    v = v.astype(jnp.bfloat16)
    return pl.pallas_call(
        _flash_body,
        grid=grid,
        in_specs=[
            pl.BlockSpec((1, _QB, _D), lambda i, qb: (i, qb, 0)),
            pl.BlockSpec((1, _NP, _D), lambda i, qb: (i, 0, 0)),
            pl.BlockSpec((1, _NP, _D), lambda i, qb: (i, 0, 0)),
            pl.BlockSpec((1, _D, _D), lambda i, qb: (i, 0, 0)),
            pl.BlockSpec((1, 1, _D), lambda i, qb: (i, 0, 0)),
        ],
        out_specs=pl.BlockSpec((1, _QB, _D), lambda i, qb: (i, qb, 0)),
        out_shape=jax.ShapeDtypeStruct((_NEI, _NP, _D), jnp.float32),
    )(q, k, v, out_w, out_b.reshape(_NEI, 1, _D))


_E = 160000
_TILES = 16
_EP = 163840
_CB = 64
_EPT = _EP // _TILES
_NCH = _EPT // _CB
_NPAD = 10240
_RPT = _NPAD // _TILES
_NSEG = 10
_SCH = _NCH // _NSEG

_SC_MESH = plsc.VectorSubcoreMesh(core_axis_name="c", subcore_axis_name="s")


def _lane_splat(vec16, j):
    return lax.gather(
        vec16, jnp.full((16, 1), j, jnp.int32),
        lax.GatherDimensionNumbers(offset_dims=(), collapsed_slice_dims=(0,),
                                   start_index_map=(0,)),
        (1,), mode=lax.GatherScatterMode.PROMISE_IN_BOUNDS)


_NSP = 4
_INTRA_MESH = plsc.VectorSubcoreMesh(core_axis_name="c", subcore_axis_name="s")
_NPT = _NPAD // _TILES
_GC = 64


@functools.partial(
    pl.kernel,
    mesh=_INTRA_MESH,
    out_type=jax.ShapeDtypeStruct((_NEI, _NPAD, _D), jnp.float32),
    scratch_types=[
        pltpu.VMEM((_NPT * _S + _GC * _S,), jnp.int32),
        pltpu.VMEM((_NPT * _S,), jnp.float32),
        pltpu.VMEM((_GC * _S, _D), jnp.float32),
        pltpu.VMEM((_GC * _S, _D), jnp.float32),
        pltpu.VMEM((_GC, _D), jnp.float32),
        pltpu.VMEM((_GC, _D), jnp.float32),
        pltpu.SemaphoreType.DMA,
        pltpu.SemaphoreType.DMA,
    ],
)
def _intra_sc(h_hbm, h0_hbm, nei_hbm, w_hbm, out_hbm,
              idx_slab, w_slab, rows0, rows1, h0_v, out_v, g0, g1):
    c = lax.axis_index("c")
    s = lax.axis_index("s")
    n0 = s * _NPT
    rows = [rows0, rows1]
    gsem = [g0, g1]
    for v in range(_GC * _S // 16):
        idx_slab[pl.ds(_NPT * _S + v * 16, 16)] = jnp.zeros((16,), jnp.int32)
    pltpu.sync_copy(nei_hbm.at[c, 0, pl.ds(n0 * _S, _NPT * _S)],
                    idx_slab.at[pl.ds(0, _NPT * _S)])
    pltpu.sync_copy(w_hbm.at[c, 0, pl.ds(n0 * _S, _NPT * _S)], w_slab)
    pltpu.async_copy(h_hbm.at[idx_slab.at[pl.ds(0, _GC * _S)]], rows[0],
                     gsem[0])

    def chunk(t, carry):
        for b in range(2):
            @pl.when(t % 2 == b)
            def _():
                pltpu.make_async_copy(
                    h_hbm.at[idx_slab.at[pl.ds(t * (_GC * _S), _GC * _S)]],
                    rows[b], gsem[b]).wait()
                pltpu.async_copy(
                    h_hbm.at[idx_slab.at[pl.ds((t + 1) * (_GC * _S),
                                               _GC * _S)]],
                    rows[1 - b], gsem[1 - b])
                pltpu.sync_copy(h0_hbm.at[pl.ds(n0 + t * _GC, _GC)], h0_v)

                def groupC(g, carry2):
                    wvs = [w_slab[pl.ds(t * (_GC * _S) + g * 80 + s5 * 16,
                                        16)]
                           for s5 in range(_S)]
                    for lj in range(16):
                        ws = [_lane_splat(wvs[s5], lj) for s5 in range(_S)]
                        r = g * 16 + lj
                        for db in range(_D // 16):
                            acc = h0_v[r, pl.ds(db * 16, 16)]
                            for s5 in range(_S):
                                acc = acc + ws[s5] * rows[b][
                                    g * 80 + s5 * 16 + lj, pl.ds(db * 16, 16)]
                            out_v[r, pl.ds(db * 16, 16)] = acc
                    return carry2
                lax.fori_loop(0, _GC // 16, groupC, 0)
                pltpu.sync_copy(out_v, out_hbm.at[c, pl.ds(n0 + t * _GC,
                                                           _GC)])
        return carry
    lax.fori_loop(0, _NPT // _GC, chunk, 0)
    nt = _NPT // _GC
    pltpu.make_async_copy(
        h_hbm.at[idx_slab.at[pl.ds(nt * (_GC * _S), _GC * _S)]],
        rows[nt % 2], gsem[nt % 2]).wait()


def _tile_order(arr):
    return arr.reshape(_NEI, _TILES, _NPT // _GC, _GC // 16, 16, _S
                       ).transpose(0, 1, 2, 3, 5, 4).reshape(
                           _NEI, 1, _NPAD * _S)


def _intra_pair(nei_h, nei_index, intra_att_w):
    h0 = nei_h[0]
    h2 = nei_h[1:].reshape(_NEI * _N, _D)
    a0 = jnp.einsum('nd,id->in', h0, intra_att_w[:, 0, :_D])
    hb = jnp.einsum('ind,id->in', nei_h[1:], intra_att_w[:, 0, _D:])
    hbg = jnp.stack([jnp.take(hb[i], nei_index[i]) for i in range(_NEI)])
    att = jax.nn.leaky_relu(a0[:, :, None] + hbg, negative_slope=0.01)
    w = jax.nn.softmax(att, axis=2)
    padn3 = ((0, 0), (0, _NPAD - _N), (0, 0))
    off = (jnp.arange(_NEI, dtype=jnp.int32) * _N)[:, None, None]
    idx = _tile_order(jnp.pad(nei_index + off, padn3))
    wso = _tile_order(jnp.pad(w, padn3))
    h0p = jnp.pad(h0, ((0, _NPAD - _N), (0, 0)))
    out = _intra_sc(h2, h0p, idx, wso)
    return out[:, :_N, :]


@functools.partial(
    pl.kernel,
    mesh=_SC_MESH,
    out_type=jax.ShapeDtypeStruct((_NSP, _NPAD, _D), jnp.float32),
    scratch_types=[
        pltpu.VMEM((_SCH + 2, _CB), jnp.int32),
        pltpu.VMEM((_SCH, _CB), jnp.int32),
        pltpu.VMEM((_SCH, _CB), jnp.float32),
        pltpu.VMEM((_CB, _D), jnp.float32),
        pltpu.VMEM((_CB, _D), jnp.float32),
        pltpu.VMEM((_CB, _D), jnp.float32),
        pltpu.VMEM_SHARED((_NPAD, _D), jnp.float32),
        pltpu.SemaphoreType.DMA,
        pltpu.SemaphoreType.DMA,
        pltpu.SemaphoreType.DMA,
        pltpu.SemaphoreType.DMA,
        pltpu.SemaphoreType.DMA,
        pltpu.SemaphoreType.DMA,
    ],
)
def _spmm4_sc(x_hbm, src_hbm, dst_hbm, w_hbm, z_hbm, out_hbm,
              src_v, dst_v, w_v, rows0, rows1, rows2, acc_sh,
              g0, g1, g2, s0, s1, s2, ):
    c = lax.axis_index("c")
    s = lax.axis_index("s")
    rows = [rows0, rows1, rows2]
    gsem = [g0, g1, g2]
    ssem = [s0, s1, s2]
    r0 = s * _RPT
    for j in range(2):
        for v in range(_CB // 16):
            src_v[_SCH + j, pl.ds(v * 16, 16)] = jnp.zeros((16,), jnp.int32)

    def scale(k, b):
        for gg in range(_CB // 16):
            wv = w_v[k, pl.ds(gg * 16, 16)]
            for j in range(16):
                wbc = _lane_splat(wv, j)
                r = gg * 16 + j
                for db in range(_D // 16):
                    rows[b][r, pl.ds(db * 16, 16)] = (
                        rows[b][r, pl.ds(db * 16, 16)] * wbc)

    def run_pass(pp, carry_p):
        p = pp * 2 + c
        pltpu.sync_copy(z_hbm, acc_sh.at[pl.ds(r0, _RPT)])
        plsc.subcore_barrier()

        def segment(g, carry0):
            pltpu.sync_copy(src_hbm.at[p, s, g], src_v.at[pl.ds(0, _SCH)])
            pltpu.sync_copy(dst_hbm.at[p, s, g], dst_v)
            pltpu.sync_copy(w_hbm.at[p, s, g], w_v)
            pltpu.async_copy(x_hbm.at[src_v.at[0]], rows[0], gsem[0])
            pltpu.async_copy(x_hbm.at[src_v.at[1]], rows[1], gsem[1])

            def chunk(k, carry):
                for b in range(3):
                    b2 = (b + 2) % 3

                    @pl.when(k % 3 == b)
                    def _():
                        pltpu.make_async_copy(x_hbm.at[src_v.at[k]],
                                              rows[b], gsem[b]).wait()
                        scale(k, b)
                        pltpu.async_copy(rows[b], acc_sh.at[dst_v.at[k]],
                                         ssem[b], add=True)

                        @pl.when(k >= 1)
                        def _():
                            pltpu.make_async_copy(
                                rows[b2], acc_sh.at[dst_v.at[k - 1]],
                                ssem[b2]).wait()
                        pltpu.async_copy(x_hbm.at[src_v.at[k + 2]],
                                         rows[b2], gsem[b2])
                return carry
            lax.fori_loop(0, _SCH, chunk, 0)
            pltpu.make_async_copy(rows[(_SCH - 1) % 3],
                                  acc_sh.at[dst_v.at[_SCH - 1]],
                                  ssem[(_SCH - 1) % 3]).wait()
            pltpu.make_async_copy(x_hbm.at[src_v.at[_SCH]],
                                  rows[_SCH % 3], gsem[_SCH % 3]).wait()
            pltpu.make_async_copy(x_hbm.at[src_v.at[_SCH + 1]],
                                  rows[(_SCH + 1) % 3],
                                  gsem[(_SCH + 1) % 3]).wait()
            return carry0
        lax.fori_loop(0, _NSEG, segment, 0)

        plsc.subcore_barrier()
        pltpu.sync_copy(acc_sh.at[pl.ds(r0, _RPT)],
                        out_hbm.at[p, pl.ds(r0, _RPT)])
        return carry_p
    lax.fori_loop(0, _NSP // 2, run_pass, 0)


def _spmm_quad(y2, x2, adj_edges, adj_w, adj_trans_edges, adj_trans_w):
    pade = ((0, 0), (0, _EP - _E))
    off = jnp.arange(_NEI, dtype=jnp.int32)[:, None] * _N
    srcs = jnp.concatenate([jnp.pad(adj_edges[:, 0, :], pade) + off,
                            jnp.pad(adj_trans_edges[:, 0, :], pade)
                            + (off + _NEI * _N)])
    dsts = jnp.concatenate([jnp.pad(adj_edges[:, 1, :], pade),
                            jnp.pad(adj_trans_edges[:, 1, :], pade)])
    ws = jnp.concatenate([jnp.pad(adj_w, pade), jnp.pad(adj_trans_w, pade)])
    sh = (_NSP, _TILES, _NSEG, _SCH, _CB)
    xall = jnp.concatenate([y2, x2]).reshape(2 * _NEI * _N, _D)
    zeros = jnp.zeros((_RPT, _D), jnp.float32)
    out = _spmm4_sc(xall, srcs.reshape(sh), dsts.reshape(sh), ws.reshape(sh),
                    zeros)
    return out[0:_NEI, :_N, :], out[_NEI:, :_N, :]


_PB = 400


def _proj_body(af_ref, w_ref, o_ref):
    o_ref[0] = jax.lax.dot_general(
        af_ref[0], w_ref[0], (((1,), (0,)), ((), ())),
        preferred_element_type=jnp.float32)


def _proj2(af2, gcn_w, att_w):
    wcat = jnp.concatenate([att_w, gcn_w], axis=2)
    out = pl.pallas_call(
        _proj_body,
        grid=(_NEI, _N // _PB),
        in_specs=[
            pl.BlockSpec((1, _PB, _D), lambda i, nb: (i, nb, 0)),
            pl.BlockSpec((1, _D, 2 * _D), lambda i, nb: (i, 0, 0)),
        ],
        out_specs=pl.BlockSpec((1, _PB, 2 * _D), lambda i, nb: (i, nb, 0)),
        out_shape=jax.ShapeDtypeStruct((_NEI, _N, 2 * _D), jnp.float32),
    )(af2, wcat)
    return out[:, :, :_D], out[:, :, _D:]


def _qkv_body(x_ref, ab_ref, w_ref, b_ref, o_ref):
    x = x_ref[0] + ab_ref[0]
    o = jnp.where(x > 0, x, jnp.exp(x) - 1.0)
    o_ref[0] = jax.lax.dot_general(
        o, w_ref[0], (((1,), (1,)), ((), ())),
        preferred_element_type=jnp.float32) + b_ref[0]


def _qkv(att_pre, att_b, in_w, in_b):
    return pl.pallas_call(
        _qkv_body,
        grid=(_NEI, _N // _PB),
        in_specs=[
            pl.BlockSpec((1, _PB, _D), lambda i, nb: (i, nb, 0)),
            pl.BlockSpec((1, 1, _D), lambda i, nb: (i, 0, 0)),
            pl.BlockSpec((1, 3 * _D, _D), lambda i, nb: (i, 0, 0)),
            pl.BlockSpec((1, 1, 3 * _D), lambda i, nb: (i, 0, 0)),
        ],
        out_specs=pl.BlockSpec((1, _PB, 3 * _D), lambda i, nb: (i, nb, 0)),
        out_shape=jax.ShapeDtypeStruct((_NEI, _N, 3 * _D), jnp.float32),
    )(att_pre, att_b.reshape(_NEI, 1, _D), in_w,
      in_b.reshape(_NEI, 1, 3 * _D))


def _inter_body(gp_ref, dif_ref, gb_ref, fcw_ref, fcb_ref, attp_ref, o_ref,
                acc_ref, m_ref, l_ref):
    nb = pl.program_id(1)
    af = gp_ref[0] + gb_ref[0]
    dif = dif_ref[0]
    t1 = jnp.tanh(jax.lax.dot_general(
        af, fcw_ref[0], (((1,), (1,)), ((), ())),
        preferred_element_type=jnp.float32) + fcb_ref[0])
    t2 = jnp.tanh(jax.lax.dot_general(
        dif, fcw_ref[0], (((1,), (1,)), ((), ())),
        preferred_element_type=jnp.float32) + fcb_ref[0])
    sp = (t1 + t2) * 0.5
    z = jax.lax.dot_general(sp, attp_ref[0], (((1,), (1,)), ((), ())),
                            preferred_element_type=jnp.float32)

    @pl.when(nb == 0)
    def _():
        m_ref[0] = -1e30
        l_ref[0] = 0.0
        acc_ref[...] = jnp.zeros_like(acc_ref)

    m_old = m_ref[0]
    m_new = jnp.maximum(m_old, jnp.max(z))
    scale = jnp.exp(m_old - m_new)
    e = jnp.exp(z - m_new)
    m_ref[0] = m_new
    l_ref[0] = l_ref[0] * scale + jnp.sum(e)
    eaf = jax.lax.dot_general(e, af, (((0,), (0,)), ((), ())),
                              preferred_element_type=jnp.float32)
    edf = jax.lax.dot_general(e, dif, (((0,), (0,)), ((), ())),
                              preferred_element_type=jnp.float32)
    acc_ref[...] = acc_ref[...] * scale + jnp.concatenate([eaf, edf], axis=0)

    @pl.when(nb == pl.num_programs(1) - 1)
    def _():
        o_ref[0] = acc_ref[...] / l_ref[0]


def _inter2(gcn_pre, dif, gcn_b, fc_w, fc_b, attp):
    return pl.pallas_call(
        _inter_body,
        grid=(_NEI, _N // _PB),
        in_specs=[
            pl.BlockSpec((1, _PB, _D), lambda i, nb: (i, nb, 0)),
            pl.BlockSpec((1, _PB, _D), lambda i, nb: (i, nb, 0)),
            pl.BlockSpec((1, 1, _D), lambda i, nb: (i, 0, 0)),
            pl.BlockSpec((1, _D, _D), lambda i, nb: (i, 0, 0)),
            pl.BlockSpec((1, 1, _D), lambda i, nb: (i, 0, 0)),
            pl.BlockSpec((1, 1, _D), lambda i, nb: (i, 0, 0)),
        ],
        out_specs=pl.BlockSpec((1, 2, _D), lambda i, nb: (i, 0, 0)),
        out_shape=jax.ShapeDtypeStruct((_NEI, 2, _D), jnp.float32),
        scratch_shapes=[
            pltpu.VMEM((2, _D), jnp.float32),
            pltpu.SMEM((1,), jnp.float32),
            pltpu.SMEM((1,), jnp.float32),
        ],
    )(gcn_pre, dif, gcn_b.reshape(_NEI, 1, _D), fc_w,
      fc_b.reshape(_NEI, 1, _D), attp)


def _spmm_x(edges, w, x):
    return jnp.zeros((_N, x.shape[1]), x.dtype).at[edges[1]].add(
        w[:, None] * x[edges[0]])


def _intra_x(nei, h, h_ref, attw):
    nei_emb = jnp.take(h, nei, axis=0)
    hr = jnp.broadcast_to(h_ref[:, None, :], nei_emb.shape)
    all_emb = jnp.concatenate([hr, nei_emb], axis=-1)
    att = jax.nn.leaky_relu(all_emb @ attw.T, negative_slope=0.01)
    att = jax.nn.softmax(att, axis=1)
    return (att * nei_emb).sum(axis=1)


def _inter_x(embeds, fcw, fcb, attp):
    sp = jnp.tanh(embeds @ fcw.T + fcb).mean(axis=1)
    beta = jax.nn.softmax(sp @ attp[0], axis=0)
    return jnp.einsum('m,mkd->kd', beta, embeds)


def kernel(nei_h, nei_index, adj_edges, adj_trans_edges, adj_w, adj_trans_w,
           intra_att_w, inter_fc_w, inter_fc_b, inter_attp, gcn_w, gcn_b,
           att_w, att_b, mha_in_w, mha_in_b, mha_out_w, mha_out_b,
           final_fc_w, final_fc_b, final_attp):
    af2 = _intra_pair(nei_h, nei_index, intra_att_w)
    y2, x2 = _proj2(af2, gcn_w, att_w)
    att_pre, gcn_pre = _spmm_quad(y2, x2, adj_edges, adj_w,
                                  adj_trans_edges, adj_trans_w)

    qkv = _qkv(att_pre, att_b, mha_in_w, mha_in_b)
    pad = ((0, 0), (0, _NP - _N), (0, 0))
    q = jnp.pad(qkv[:, :, :_D], pad)
    k = jnp.pad(qkv[:, :, _D:2 * _D], pad)
    v = jnp.pad(qkv[:, :, 2 * _D:], pad)
    dif = _flash_attention(q, k, v, mha_out_w, mha_out_b)[:, :_N, :]

    multi = _inter2(gcn_pre, dif, gcn_b, inter_fc_w, inter_fc_b, inter_attp)
    final_in = multi.transpose(1, 0, 2)
    return _inter_x(final_in, final_fc_w, final_fc_b, final_attp)


# speedup vs baseline: 2.6837x; 2.6837x over previous
import functools

import jax
import jax.numpy as jnp
import numpy as np
from jax import lax
from jax.experimental import pallas as pl
from jax.experimental.pallas import tpu as pltpu
from jax.experimental.pallas import tpu_sc as plsc

_N = 10000
_D = 128
_NEI = 2
_S = 5
_NP = 10240
_QB = 256


def _flash_body(q_ref, k_ref, v_ref, ow_ref, ob_ref, o_ref):
    q = q_ref[0]
    k = k_ref[0]
    v = v_ref[0]
    s = jax.lax.dot_general(q, k, (((1,), (1,)), ((), ())),
                            preferred_element_type=jnp.float32)
    s = s * (1.0 / np.sqrt(_D).astype(np.float32))
    kidx = jax.lax.broadcasted_iota(jnp.int32, s.shape, 1)
    s = jnp.where(kidx < _N, s, -1e30)
    m = jnp.max(s, axis=-1, keepdims=True)
    p = jnp.exp(s - m)
    l = jnp.sum(p, axis=-1, keepdims=True)
    o = jax.lax.dot_general(p.astype(jnp.bfloat16), v, (((1,), (0,)), ((), ())),
                            preferred_element_type=jnp.float32) / l
    o = jax.lax.dot_general(o, ow_ref[0], (((1,), (1,)), ((), ())),
                            preferred_element_type=jnp.float32)
    o_ref[0] = o + ob_ref[0]


def _flash_attention(q, k, v, out_w, out_b):
    grid = (_NEI, _NP // _QB)
    q = q.astype(jnp.bfloat16)
    k = k.astype(jnp.bfloat16)
    v = v.astype(jnp.bfloat16)
    return pl.pallas_call(
        _flash_body,
        grid=grid,
        in_specs=[
            pl.BlockSpec((1, _QB, _D), lambda i, qb: (i, qb, 0)),
            pl.BlockSpec((1, _NP, _D), lambda i, qb: (i, 0, 0)),
            pl.BlockSpec((1, _NP, _D), lambda i, qb: (i, 0, 0)),
            pl.BlockSpec((1, _D, _D), lambda i, qb: (i, 0, 0)),
            pl.BlockSpec((1, 1, _D), lambda i, qb: (i, 0, 0)),
        ],
        out_specs=pl.BlockSpec((1, _QB, _D), lambda i, qb: (i, qb, 0)),
        out_shape=jax.ShapeDtypeStruct((_NEI, _NP, _D), jnp.float32),
    )(q, k, v, out_w, out_b.reshape(_NEI, 1, _D))


_E = 160000
_TILES = 16
_CB = 80
_EPT = _E // _TILES
_NCH = _EPT // _CB
_NPAD = 10240
_RPT = _NPAD // _TILES
_ZR = 40
_NSEG = 5
_SCH = _NCH // _NSEG

_SC_MESH = plsc.VectorSubcoreMesh(core_axis_name="c", subcore_axis_name="s")


def _lane_splat(vec16, j):
    return lax.gather(
        vec16, jnp.full((16, 1), j, jnp.int32),
        lax.GatherDimensionNumbers(offset_dims=(), collapsed_slice_dims=(0,),
                                   start_index_map=(0,)),
        (1,), mode=lax.GatherScatterMode.PROMISE_IN_BOUNDS)


_NSP = 4
_INTRA_MESH = plsc.VectorSubcoreMesh(core_axis_name="c", subcore_axis_name="s")
_NPT = _NPAD // _TILES
_GC = 64


@functools.partial(
    pl.kernel,
    mesh=_INTRA_MESH,
    out_type=jax.ShapeDtypeStruct((_NEI, _NPAD, _D), jnp.float32),
    scratch_types=[
        pltpu.VMEM((_NPT * _S,), jnp.int32),
        pltpu.VMEM((_NPT * _S,), jnp.float32),
        pltpu.VMEM((_GC * _S, _D), jnp.float32),
        pltpu.VMEM((_GC, _D), jnp.float32),
        pltpu.VMEM((_GC, _D), jnp.float32),
        pltpu.SemaphoreType.DMA,
    ],
)
def _intra_sc(h_hbm, h0_hbm, nei_hbm, w_hbm, out_hbm,
              idx_slab, w_slab, rows_v, h0_v, out_v, sem):
    c = lax.axis_index("c")
    s = lax.axis_index("s")
    n0 = s * _NPT
    pltpu.sync_copy(nei_hbm.at[c, 0, pl.ds(n0 * _S, _NPT * _S)], idx_slab)
    pltpu.sync_copy(w_hbm.at[c, 0, pl.ds(n0 * _S, _NPT * _S)], w_slab)

    def chunk(t, carry):
        pltpu.async_copy(h_hbm.at[idx_slab.at[pl.ds(t * (_GC * _S),
                                                    _GC * _S)]],
                         rows_v, sem).wait()
        pltpu.sync_copy(h0_hbm.at[pl.ds(n0 + t * _GC, _GC)], h0_v)

        def groupC(g, carry2):
            wvs = [w_slab[pl.ds(t * (_GC * _S) + g * 80 + s5 * 16, 16)]
                   for s5 in range(_S)]
            for lj in range(16):
                ws = [_lane_splat(wvs[s5], lj) for s5 in range(_S)]
                r = g * 16 + lj
                for db in range(_D // 16):
                    acc = h0_v[r, pl.ds(db * 16, 16)]
                    for s5 in range(_S):
                        acc = acc + ws[s5] * rows_v[g * 80 + s5 * 16 + lj,
                                                    pl.ds(db * 16, 16)]
                    out_v[r, pl.ds(db * 16, 16)] = acc
            return carry2
        lax.fori_loop(0, _GC // 16, groupC, 0)
        pltpu.sync_copy(out_v, out_hbm.at[c, pl.ds(n0 + t * _GC, _GC)])
        return carry
    lax.fori_loop(0, _NPT // _GC, chunk, 0)


def _tile_order(arr):
    return arr.reshape(_NEI, _TILES, _NPT // _GC, _GC // 16, 16, _S
                       ).transpose(0, 1, 2, 3, 5, 4).reshape(
                           _NEI, 1, _NPAD * _S)


def _intra_pair(nei_h, nei_index, intra_att_w):
    h0 = nei_h[0]
    h2 = nei_h[1:].reshape(_NEI * _N, _D)
    a0 = jnp.einsum('nd,id->in', h0, intra_att_w[:, 0, :_D])
    hb = jnp.einsum('ind,id->in', nei_h[1:], intra_att_w[:, 0, _D:])
    hbg = jnp.stack([jnp.take(hb[i], nei_index[i]) for i in range(_NEI)])
    att = jax.nn.leaky_relu(a0[:, :, None] + hbg, negative_slope=0.01)
    w = jax.nn.softmax(att, axis=2)
    padn3 = ((0, 0), (0, _NPAD - _N), (0, 0))
    off = (jnp.arange(_NEI, dtype=jnp.int32) * _N)[:, None, None]
    idx = _tile_order(jnp.pad(nei_index + off, padn3))
    wso = _tile_order(jnp.pad(w, padn3))
    h0p = jnp.pad(h0, ((0, _NPAD - _N), (0, 0)))
    out = _intra_sc(h2, h0p, idx, wso)
    return out[:, :_N, :]


@functools.partial(
    pl.kernel,
    mesh=_SC_MESH,
    out_type=jax.ShapeDtypeStruct((_NSP, _NPAD, _D), jnp.float32),
    scratch_types=[
        pltpu.VMEM((_SCH, _CB), jnp.int32),
        pltpu.VMEM((_SCH, _CB), jnp.int32),
        pltpu.VMEM((_SCH, _CB), jnp.float32),
        pltpu.VMEM((_CB, _D), jnp.float32),
        pltpu.VMEM((_ZR, _D), jnp.float32),
        pltpu.VMEM_SHARED((_NPAD, _D), jnp.float32),
        pltpu.SemaphoreType.DMA,
    ],
)
def _spmm4_sc(x_hbm, src_hbm, dst_hbm, w_hbm, out_hbm,
              src_v, dst_v, w_v, rows_v, z_v, acc_sh, sem):
    c = lax.axis_index("c")
    s = lax.axis_index("s")

    def zrow(r, carry):
        for db in range(_D // 16):
            z_v[r, pl.ds(db * 16, 16)] = jnp.zeros((16,), jnp.float32)
        return carry
    lax.fori_loop(0, _ZR, zrow, 0)
    r0 = s * _RPT

    for pp in range(_NSP // 2):
        p = pp * 2 + c
        for kz in range(_RPT // _ZR):
            pltpu.sync_copy(z_v, acc_sh.at[pl.ds(r0 + kz * _ZR, _ZR)])
        plsc.subcore_barrier()

        def segment(g, carry0):
            pltpu.sync_copy(src_hbm.at[p, s, g], src_v)
            pltpu.sync_copy(dst_hbm.at[p, s, g], dst_v)
            pltpu.sync_copy(w_hbm.at[p, s, g], w_v)

            def chunk(k, carry):
                pltpu.async_copy(x_hbm.at[src_v.at[k]], rows_v, sem).wait()

                def sgroup(gg, carry2):
                    wv = w_v[k, pl.ds(gg * 16, 16)]
                    for j in range(16):
                        wbc = _lane_splat(wv, j)
                        r = gg * 16 + j
                        for db in range(_D // 16):
                            rows_v[r, pl.ds(db * 16, 16)] = (
                                rows_v[r, pl.ds(db * 16, 16)] * wbc)
                    return carry2
                lax.fori_loop(0, _CB // 16, sgroup, 0)
                pltpu.sync_copy(rows_v, acc_sh.at[dst_v.at[k]], add=True)
                return carry
            lax.fori_loop(0, _SCH, chunk, 0)
            return carry0
        lax.fori_loop(0, _NSEG, segment, 0)

        plsc.subcore_barrier()
        pltpu.sync_copy(acc_sh.at[pl.ds(r0, _RPT)],
                        out_hbm.at[p, pl.ds(r0, _RPT)])


def _spmm_quad(y2, x2, adj_edges, adj_w, adj_trans_edges, adj_trans_w):
    off = jnp.arange(_NEI, dtype=jnp.int32)[:, None] * _N
    srcs = jnp.concatenate([adj_edges[:, 0, :] + off,
                            adj_trans_edges[:, 0, :] + (off + _NEI * _N)])
    dsts = jnp.concatenate([adj_edges[:, 1, :], adj_trans_edges[:, 1, :]])
    ws = jnp.concatenate([adj_w, adj_trans_w])
    sh = (_NSP, _TILES, _NSEG, _SCH, _CB)
    xall = jnp.concatenate([y2, x2]).reshape(2 * _NEI * _N, _D)
    out = _spmm4_sc(xall, srcs.reshape(sh), dsts.reshape(sh), ws.reshape(sh))
    return out[0:_NEI, :_N, :], out[_NEI:, :_N, :]


_PB = 400


def _proj_body(af_ref, w_ref, o_ref):
    o_ref[0] = jax.lax.dot_general(
        af_ref[0], w_ref[0], (((1,), (0,)), ((), ())),
        preferred_element_type=jnp.float32)


def _proj2(af2, gcn_w, att_w):
    wcat = jnp.concatenate([att_w, gcn_w], axis=2)
    out = pl.pallas_call(
        _proj_body,
        grid=(_NEI, _N // _PB),
        in_specs=[
            pl.BlockSpec((1, _PB, _D), lambda i, nb: (i, nb, 0)),
            pl.BlockSpec((1, _D, 2 * _D), lambda i, nb: (i, 0, 0)),
        ],
        out_specs=pl.BlockSpec((1, _PB, 2 * _D), lambda i, nb: (i, nb, 0)),
        out_shape=jax.ShapeDtypeStruct((_NEI, _N, 2 * _D), jnp.float32),
    )(af2, wcat)
    return out[:, :, :_D], out[:, :, _D:]


def _qkv_body(x_ref, ab_ref, w_ref, b_ref, o_ref):
    x = x_ref[0] + ab_ref[0]
    o = jnp.where(x > 0, x, jnp.exp(x) - 1.0)
    o_ref[0] = jax.lax.dot_general(
        o, w_ref[0], (((1,), (1,)), ((), ())),
        preferred_element_type=jnp.float32) + b_ref[0]


def _qkv(att_pre, att_b, in_w, in_b):
    return pl.pallas_call(
        _qkv_body,
        grid=(_NEI, _N // _PB),
        in_specs=[
            pl.BlockSpec((1, _PB, _D), lambda i, nb: (i, nb, 0)),
            pl.BlockSpec((1, 1, _D), lambda i, nb: (i, 0, 0)),
            pl.BlockSpec((1, 3 * _D, _D), lambda i, nb: (i, 0, 0)),
            pl.BlockSpec((1, 1, 3 * _D), lambda i, nb: (i, 0, 0)),
        ],
        out_specs=pl.BlockSpec((1, _PB, 3 * _D), lambda i, nb: (i, nb, 0)),
        out_shape=jax.ShapeDtypeStruct((_NEI, _N, 3 * _D), jnp.float32),
    )(att_pre, att_b.reshape(_NEI, 1, _D), in_w,
      in_b.reshape(_NEI, 1, 3 * _D))


def _inter_body(gp_ref, dif_ref, gb_ref, fcw_ref, fcb_ref, attp_ref, o_ref,
                acc_ref, m_ref, l_ref):
    nb = pl.program_id(1)
    af = gp_ref[0] + gb_ref[0]
    dif = dif_ref[0]
    t1 = jnp.tanh(jax.lax.dot_general(
        af, fcw_ref[0], (((1,), (1,)), ((), ())),
        preferred_element_type=jnp.float32) + fcb_ref[0])
    t2 = jnp.tanh(jax.lax.dot_general(
        dif, fcw_ref[0], (((1,), (1,)), ((), ())),
        preferred_element_type=jnp.float32) + fcb_ref[0])
    sp = (t1 + t2) * 0.5
    z = jax.lax.dot_general(sp, attp_ref[0], (((1,), (1,)), ((), ())),
                            preferred_element_type=jnp.float32)

    @pl.when(nb == 0)
    def _():
        m_ref[0] = -1e30
        l_ref[0] = 0.0
        acc_ref[...] = jnp.zeros_like(acc_ref)

    m_old = m_ref[0]
    m_new = jnp.maximum(m_old, jnp.max(z))
    scale = jnp.exp(m_old - m_new)
    e = jnp.exp(z - m_new)
    m_ref[0] = m_new
    l_ref[0] = l_ref[0] * scale + jnp.sum(e)
    eaf = jax.lax.dot_general(e, af, (((0,), (0,)), ((), ())),
                              preferred_element_type=jnp.float32)
    edf = jax.lax.dot_general(e, dif, (((0,), (0,)), ((), ())),
                              preferred_element_type=jnp.float32)
    acc_ref[...] = acc_ref[...] * scale + jnp.concatenate([eaf, edf], axis=0)

    @pl.when(nb == pl.num_programs(1) - 1)
    def _():
        o_ref[0] = acc_ref[...] / l_ref[0]


def _inter2(gcn_pre, dif, gcn_b, fc_w, fc_b, attp):
    return pl.pallas_call(
        _inter_body,
        grid=(_NEI, _N // _PB),
        in_specs=[
            pl.BlockSpec((1, _PB, _D), lambda i, nb: (i, nb, 0)),
            pl.BlockSpec((1, _PB, _D), lambda i, nb: (i, nb, 0)),
            pl.BlockSpec((1, 1, _D), lambda i, nb: (i, 0, 0)),
            pl.BlockSpec((1, _D, _D), lambda i, nb: (i, 0, 0)),
            pl.BlockSpec((1, 1, _D), lambda i, nb: (i, 0, 0)),
            pl.BlockSpec((1, 1, _D), lambda i, nb: (i, 0, 0)),
        ],
        out_specs=pl.BlockSpec((1, 2, _D), lambda i, nb: (i, 0, 0)),
        out_shape=jax.ShapeDtypeStruct((_NEI, 2, _D), jnp.float32),
        scratch_shapes=[
            pltpu.VMEM((2, _D), jnp.float32),
            pltpu.SMEM((1,), jnp.float32),
            pltpu.SMEM((1,), jnp.float32),
        ],
    )(gcn_pre, dif, gcn_b.reshape(_NEI, 1, _D), fc_w,
      fc_b.reshape(_NEI, 1, _D), attp)


def _spmm_x(edges, w, x):
    return jnp.zeros((_N, x.shape[1]), x.dtype).at[edges[1]].add(
        w[:, None] * x[edges[0]])


def _intra_x(nei, h, h_ref, attw):
    nei_emb = jnp.take(h, nei, axis=0)
    hr = jnp.broadcast_to(h_ref[:, None, :], nei_emb.shape)
    all_emb = jnp.concatenate([hr, nei_emb], axis=-1)
    att = jax.nn.leaky_relu(all_emb @ attw.T, negative_slope=0.01)
    att = jax.nn.softmax(att, axis=1)
    return (att * nei_emb).sum(axis=1)


def _inter_x(embeds, fcw, fcb, attp):
    sp = jnp.tanh(embeds @ fcw.T + fcb).mean(axis=1)
    beta = jax.nn.softmax(sp @ attp[0], axis=0)
    return jnp.einsum('m,mkd->kd', beta, embeds)


def kernel(nei_h, nei_index, adj_edges, adj_trans_edges, adj_w, adj_trans_w,
           intra_att_w, inter_fc_w, inter_fc_b, inter_attp, gcn_w, gcn_b,
           att_w, att_b, mha_in_w, mha_in_b, mha_out_w, mha_out_b,
           final_fc_w, final_fc_b, final_attp):
    af2 = _intra_pair(nei_h, nei_index, intra_att_w)
    y2, x2 = _proj2(af2, gcn_w, att_w)
    att_pre, gcn_pre = _spmm_quad(y2, x2, adj_edges, adj_w,
                                  adj_trans_edges, adj_trans_w)

    qkv = _qkv(att_pre, att_b, mha_in_w, mha_in_b)
    pad = ((0, 0), (0, _NP - _N), (0, 0))
    q = jnp.pad(qkv[:, :, :_D], pad)
    k = jnp.pad(qkv[:, :, _D:2 * _D], pad)
    v = jnp.pad(qkv[:, :, 2 * _D:], pad)
    dif = _flash_attention(q, k, v, mha_out_w, mha_out_b)[:, :_N, :]

    multi = _inter2(gcn_pre, dif, gcn_b, inter_fc_w, inter_fc_b, inter_attp)
    final_in = multi.transpose(1, 0, 2)
    return _inter_x(final_in, final_fc_w, final_fc_b, final_attp)


# speedup vs baseline: 2.7880x; 1.0389x over previous
import functools

import jax
import jax.numpy as jnp
import numpy as np
from jax import lax
from jax.experimental import pallas as pl
from jax.experimental.pallas import tpu as pltpu
from jax.experimental.pallas import tpu_sc as plsc

_N = 10000
_D = 128
_NEI = 2
_S = 5
_NP = 10240
_QB = 512


def _flash_body(q_ref, k_ref, v_ref, ow_ref, ob_ref, o_ref):
    q = q_ref[0]
    k = k_ref[0]
    v = v_ref[0]
    s = jax.lax.dot_general(q, k, (((1,), (1,)), ((), ())),
                            preferred_element_type=jnp.float32)
    s = s * (1.0 / np.sqrt(_D).astype(np.float32))
    kidx = jax.lax.broadcasted_iota(jnp.int32, s.shape, 1)
    s = jnp.where(kidx < _N, s, -1e30)
    m = jnp.max(s, axis=-1, keepdims=True)
    p = jnp.exp(s - m)
    l = jnp.sum(p, axis=-1, keepdims=True)
    o = jax.lax.dot_general(p.astype(jnp.bfloat16), v, (((1,), (0,)), ((), ())),
                            preferred_element_type=jnp.float32) / l
    o = jax.lax.dot_general(o, ow_ref[0], (((1,), (1,)), ((), ())),
                            preferred_element_type=jnp.float32)
    o_ref[0] = o + ob_ref[0]


def _flash_attention(q, k, v, out_w, out_b):
    grid = (_NEI, _NP // _QB)
    q = q.astype(jnp.bfloat16)
    k = k.astype(jnp.bfloat16)
    v = v.astype(jnp.bfloat16)
    return pl.pallas_call(
        _flash_body,
        grid=grid,
        in_specs=[
            pl.BlockSpec((1, _QB, _D), lambda i, qb: (i, qb, 0)),
            pl.BlockSpec((1, _NP, _D), lambda i, qb: (i, 0, 0)),
            pl.BlockSpec((1, _NP, _D), lambda i, qb: (i, 0, 0)),
            pl.BlockSpec((1, _D, _D), lambda i, qb: (i, 0, 0)),
            pl.BlockSpec((1, 1, _D), lambda i, qb: (i, 0, 0)),
        ],
        out_specs=pl.BlockSpec((1, _QB, _D), lambda i, qb: (i, qb, 0)),
        out_shape=jax.ShapeDtypeStruct((_NEI, _NP, _D), jnp.float32),
    )(q, k, v, out_w, out_b.reshape(_NEI, 1, _D))


_E = 160000
_TILES = 16
_CB = 80
_EPT = _E // _TILES
_NCH = _EPT // _CB
_NPAD = 10240
_RPT = _NPAD // _TILES
_ZR = 40
_NSEG = 5
_SCH = _NCH // _NSEG

_SC_MESH = plsc.VectorSubcoreMesh(core_axis_name="c", subcore_axis_name="s")


def _lane_splat(vec16, j):
    return lax.gather(
        vec16, jnp.full((16, 1), j, jnp.int32),
        lax.GatherDimensionNumbers(offset_dims=(), collapsed_slice_dims=(0,),
                                   start_index_map=(0,)),
        (1,), mode=lax.GatherScatterMode.PROMISE_IN_BOUNDS)


_NSP = 4
_INTRA_MESH = plsc.VectorSubcoreMesh(core_axis_name="c", subcore_axis_name="s")
_NPT = _NPAD // _TILES
_GC = 64


@functools.partial(
    pl.kernel,
    mesh=_INTRA_MESH,
    out_type=jax.ShapeDtypeStruct((_NEI, _NPAD, _D), jnp.float32),
    scratch_types=[
        pltpu.VMEM((_NPT * _S,), jnp.int32),
        pltpu.VMEM((_NPT * _S,), jnp.float32),
        pltpu.VMEM((_GC * _S, _D), jnp.float32),
        pltpu.VMEM((_GC, _D), jnp.float32),
        pltpu.VMEM((_GC, _D), jnp.float32),
        pltpu.SemaphoreType.DMA,
    ],
)
def _intra_sc(h_hbm, h0_hbm, nei_hbm, w_hbm, out_hbm,
              idx_slab, w_slab, rows_v, h0_v, out_v, sem):
    c = lax.axis_index("c")
    s = lax.axis_index("s")
    n0 = s * _NPT
    pltpu.sync_copy(nei_hbm.at[c, 0, pl.ds(n0 * _S, _NPT * _S)], idx_slab)
    pltpu.sync_copy(w_hbm.at[c, 0, pl.ds(n0 * _S, _NPT * _S)], w_slab)

    def chunk(t, carry):
        pltpu.async_copy(h_hbm.at[idx_slab.at[pl.ds(t * (_GC * _S),
                                                    _GC * _S)]],
                         rows_v, sem).wait()
        pltpu.sync_copy(h0_hbm.at[pl.ds(n0 + t * _GC, _GC)], h0_v)

        def groupC(g, carry2):
            wvs = [w_slab[pl.ds(t * (_GC * _S) + g * 80 + s5 * 16, 16)]
                   for s5 in range(_S)]
            for lj in range(16):
                ws = [_lane_splat(wvs[s5], lj) for s5 in range(_S)]
                r = g * 16 + lj
                for db in range(_D // 16):
                    acc = h0_v[r, pl.ds(db * 16, 16)]
                    for s5 in range(_S):
                        acc = acc + ws[s5] * rows_v[g * 80 + s5 * 16 + lj,
                                                    pl.ds(db * 16, 16)]
                    out_v[r, pl.ds(db * 16, 16)] = acc
            return carry2
        lax.fori_loop(0, _GC // 16, groupC, 0)
        pltpu.sync_copy(out_v, out_hbm.at[c, pl.ds(n0 + t * _GC, _GC)])
        return carry
    lax.fori_loop(0, _NPT // _GC, chunk, 0)


def _tile_order(arr):
    return arr.reshape(_NEI, _TILES, _NPT // _GC, _GC // 16, 16, _S
                       ).transpose(0, 1, 2, 3, 5, 4).reshape(
                           _NEI, 1, _NPAD * _S)


def _intra_pair(nei_h, nei_index, intra_att_w):
    h0 = nei_h[0]
    h2 = nei_h[1:].reshape(_NEI * _N, _D)
    a0 = jnp.einsum('nd,id->in', h0, intra_att_w[:, 0, :_D])
    hb = jnp.einsum('ind,id->in', nei_h[1:], intra_att_w[:, 0, _D:])
    hbg = jnp.stack([jnp.take(hb[i], nei_index[i]) for i in range(_NEI)])
    att = jax.nn.leaky_relu(a0[:, :, None] + hbg, negative_slope=0.01)
    w = jax.nn.softmax(att, axis=2)
    padn3 = ((0, 0), (0, _NPAD - _N), (0, 0))
    off = (jnp.arange(_NEI, dtype=jnp.int32) * _N)[:, None, None]
    idx = _tile_order(jnp.pad(nei_index + off, padn3))
    wso = _tile_order(jnp.pad(w, padn3))
    h0p = jnp.pad(h0, ((0, _NPAD - _N), (0, 0)))
    out = _intra_sc(h2, h0p, idx, wso)
    return out[:, :_N, :]


@functools.partial(
    pl.kernel,
    mesh=_SC_MESH,
    out_type=jax.ShapeDtypeStruct((_NEI, _NPAD, _D), jnp.float32),
    scratch_types=[
        pltpu.VMEM((_SCH, _CB), jnp.int32),
        pltpu.VMEM((_SCH, _CB), jnp.int32),
        pltpu.VMEM((_SCH, _CB), jnp.float32),
        pltpu.VMEM((_CB, _D), jnp.float32),
        pltpu.VMEM((_ZR, _D), jnp.float32),
        pltpu.VMEM_SHARED((_NPAD, _D), jnp.float32),
        pltpu.SemaphoreType.DMA,
    ],
)
def _spmm2_sc(x_hbm, src_hbm, dst_hbm, w_hbm, out_hbm,
              src_v, dst_v, w_v, rows_v, z_v, acc_sh, sem):
    c = lax.axis_index("c")
    s = lax.axis_index("s")

    def zrow(r, carry):
        for db in range(_D // 16):
            z_v[r, pl.ds(db * 16, 16)] = jnp.zeros((16,), jnp.float32)
        return carry
    lax.fori_loop(0, _ZR, zrow, 0)
    r0 = s * _RPT

    for kz in range(_RPT // _ZR):
        pltpu.sync_copy(z_v, acc_sh.at[pl.ds(r0 + kz * _ZR, _ZR)])
    plsc.subcore_barrier()

    def segment(g, carry0):
        pltpu.sync_copy(src_hbm.at[c, s, g], src_v)
        pltpu.sync_copy(dst_hbm.at[c, s, g], dst_v)
        pltpu.sync_copy(w_hbm.at[c, s, g], w_v)

        def chunk(k, carry):
            pltpu.async_copy(x_hbm.at[src_v.at[k]], rows_v, sem).wait()

            def sgroup(gg, carry2):
                wv = w_v[k, pl.ds(gg * 16, 16)]
                for j in range(16):
                    wbc = _lane_splat(wv, j)
                    r = gg * 16 + j
                    for db in range(_D // 16):
                        rows_v[r, pl.ds(db * 16, 16)] = (
                            rows_v[r, pl.ds(db * 16, 16)] * wbc)
                return carry2
            lax.fori_loop(0, _CB // 16, sgroup, 0)
            pltpu.sync_copy(rows_v, acc_sh.at[dst_v.at[k]], add=True)
            return carry
        lax.fori_loop(0, _SCH, chunk, 0)
        return carry0
    lax.fori_loop(0, _NSEG, segment, 0)

    plsc.subcore_barrier()
    pltpu.sync_copy(acc_sh.at[pl.ds(r0, _RPT)],
                    out_hbm.at[c, pl.ds(r0, _RPT)])


def _spmm_pair(x2, edges2, w2):
    off = jnp.arange(_NEI, dtype=jnp.int32)[:, None] * _N
    srcs = edges2[:, 0, :] + off
    dsts = edges2[:, 1, :]
    sh = (_NEI, _TILES, _NSEG, _SCH, _CB)
    out = _spmm2_sc(x2.reshape(_NEI * _N, _D), srcs.reshape(sh),
                    dsts.reshape(sh), w2.reshape(sh))
    return out[:, :_N, :]


_PB = 400


def _proj_body(af_ref, w_ref, o_ref):
    o_ref[0] = jax.lax.dot_general(
        af_ref[0], w_ref[0], (((1,), (0,)), ((), ())),
        preferred_element_type=jnp.float32)


def _proj2(af2, gcn_w, att_w):
    wcat = jnp.concatenate([att_w, gcn_w], axis=2)
    out = pl.pallas_call(
        _proj_body,
        grid=(_NEI, _N // _PB),
        in_specs=[
            pl.BlockSpec((1, _PB, _D), lambda i, nb: (i, nb, 0)),
            pl.BlockSpec((1, _D, 2 * _D), lambda i, nb: (i, 0, 0)),
        ],
        out_specs=pl.BlockSpec((1, _PB, 2 * _D), lambda i, nb: (i, nb, 0)),
        out_shape=jax.ShapeDtypeStruct((_NEI, _N, 2 * _D), jnp.float32),
    )(af2, wcat)
    return out[:, :, :_D], out[:, :, _D:]


def _qkv_body(x_ref, ab_ref, w_ref, b_ref, o_ref):
    x = x_ref[0] + ab_ref[0]
    o = jnp.where(x > 0, x, jnp.exp(x) - 1.0)
    o_ref[0] = jax.lax.dot_general(
        o, w_ref[0], (((1,), (1,)), ((), ())),
        preferred_element_type=jnp.float32) + b_ref[0]


def _qkv(att_pre, att_b, in_w, in_b):
    return pl.pallas_call(
        _qkv_body,
        grid=(_NEI, _N // _PB),
        in_specs=[
            pl.BlockSpec((1, _PB, _D), lambda i, nb: (i, nb, 0)),
            pl.BlockSpec((1, 1, _D), lambda i, nb: (i, 0, 0)),
            pl.BlockSpec((1, 3 * _D, _D), lambda i, nb: (i, 0, 0)),
            pl.BlockSpec((1, 1, 3 * _D), lambda i, nb: (i, 0, 0)),
        ],
        out_specs=pl.BlockSpec((1, _PB, 3 * _D), lambda i, nb: (i, nb, 0)),
        out_shape=jax.ShapeDtypeStruct((_NEI, _N, 3 * _D), jnp.float32),
    )(att_pre, att_b.reshape(_NEI, 1, _D), in_w,
      in_b.reshape(_NEI, 1, 3 * _D))


def _inter_body(gp_ref, dif_ref, gb_ref, fcw_ref, fcb_ref, attp_ref, o_ref,
                acc_ref, m_ref, l_ref):
    nb = pl.program_id(1)
    af = gp_ref[0] + gb_ref[0]
    dif = dif_ref[0]
    t1 = jnp.tanh(jax.lax.dot_general(
        af, fcw_ref[0], (((1,), (1,)), ((), ())),
        preferred_element_type=jnp.float32) + fcb_ref[0])
    t2 = jnp.tanh(jax.lax.dot_general(
        dif, fcw_ref[0], (((1,), (1,)), ((), ())),
        preferred_element_type=jnp.float32) + fcb_ref[0])
    sp = (t1 + t2) * 0.5
    z = jax.lax.dot_general(sp, attp_ref[0], (((1,), (1,)), ((), ())),
                            preferred_element_type=jnp.float32)

    @pl.when(nb == 0)
    def _():
        m_ref[0] = -1e30
        l_ref[0] = 0.0
        acc_ref[...] = jnp.zeros_like(acc_ref)

    m_old = m_ref[0]
    m_new = jnp.maximum(m_old, jnp.max(z))
    scale = jnp.exp(m_old - m_new)
    e = jnp.exp(z - m_new)
    m_ref[0] = m_new
    l_ref[0] = l_ref[0] * scale + jnp.sum(e)
    eaf = jax.lax.dot_general(e, af, (((0,), (0,)), ((), ())),
                              preferred_element_type=jnp.float32)
    edf = jax.lax.dot_general(e, dif, (((0,), (0,)), ((), ())),
                              preferred_element_type=jnp.float32)
    acc_ref[...] = acc_ref[...] * scale + jnp.concatenate([eaf, edf], axis=0)

    @pl.when(nb == pl.num_programs(1) - 1)
    def _():
        o_ref[0] = acc_ref[...] / l_ref[0]


def _inter2(gcn_pre, dif, gcn_b, fc_w, fc_b, attp):
    return pl.pallas_call(
        _inter_body,
        grid=(_NEI, _N // _PB),
        in_specs=[
            pl.BlockSpec((1, _PB, _D), lambda i, nb: (i, nb, 0)),
            pl.BlockSpec((1, _PB, _D), lambda i, nb: (i, nb, 0)),
            pl.BlockSpec((1, 1, _D), lambda i, nb: (i, 0, 0)),
            pl.BlockSpec((1, _D, _D), lambda i, nb: (i, 0, 0)),
            pl.BlockSpec((1, 1, _D), lambda i, nb: (i, 0, 0)),
            pl.BlockSpec((1, 1, _D), lambda i, nb: (i, 0, 0)),
        ],
        out_specs=pl.BlockSpec((1, 2, _D), lambda i, nb: (i, 0, 0)),
        out_shape=jax.ShapeDtypeStruct((_NEI, 2, _D), jnp.float32),
        scratch_shapes=[
            pltpu.VMEM((2, _D), jnp.float32),
            pltpu.SMEM((1,), jnp.float32),
            pltpu.SMEM((1,), jnp.float32),
        ],
    )(gcn_pre, dif, gcn_b.reshape(_NEI, 1, _D), fc_w,
      fc_b.reshape(_NEI, 1, _D), attp)


def _spmm_x(edges, w, x):
    return jnp.zeros((_N, x.shape[1]), x.dtype).at[edges[1]].add(
        w[:, None] * x[edges[0]])


def _intra_x(nei, h, h_ref, attw):
    nei_emb = jnp.take(h, nei, axis=0)
    hr = jnp.broadcast_to(h_ref[:, None, :], nei_emb.shape)
    all_emb = jnp.concatenate([hr, nei_emb], axis=-1)
    att = jax.nn.leaky_relu(all_emb @ attw.T, negative_slope=0.01)
    att = jax.nn.softmax(att, axis=1)
    return (att * nei_emb).sum(axis=1)


def _inter_x(embeds, fcw, fcb, attp):
    sp = jnp.tanh(embeds @ fcw.T + fcb).mean(axis=1)
    beta = jax.nn.softmax(sp @ attp[0], axis=0)
    return jnp.einsum('m,mkd->kd', beta, embeds)


def kernel(nei_h, nei_index, adj_edges, adj_trans_edges, adj_w, adj_trans_w,
           intra_att_w, inter_fc_w, inter_fc_b, inter_attp, gcn_w, gcn_b,
           att_w, att_b, mha_in_w, mha_in_b, mha_out_w, mha_out_b,
           final_fc_w, final_fc_b, final_attp):
    af2 = _intra_pair(nei_h, nei_index, intra_att_w)
    y2, x2 = _proj2(af2, gcn_w, att_w)
    att_pre = _spmm_pair(y2, adj_edges, adj_w)
    gcn_pre = _spmm_pair(x2, adj_trans_edges, adj_trans_w)

    qkv = _qkv(att_pre, att_b, mha_in_w, mha_in_b)
    pad = ((0, 0), (0, _NP - _N), (0, 0))
    q = jnp.pad(qkv[:, :, :_D], pad)
    k = jnp.pad(qkv[:, :, _D:2 * _D], pad)
    v = jnp.pad(qkv[:, :, 2 * _D:], pad)
    dif = _flash_attention(q, k, v, mha_out_w, mha_out_b)[:, :_N, :]

    multi = _inter2(gcn_pre, dif, gcn_b, inter_fc_w, inter_fc_b, inter_attp)
    final_in = multi.transpose(1, 0, 2)
    return _inter_x(final_in, final_fc_w, final_fc_b, final_attp)


# speedup vs baseline: 2.8763x; 1.0317x over previous
import functools

import jax
import jax.numpy as jnp
import numpy as np
from jax import lax
from jax.experimental import pallas as pl
from jax.experimental.pallas import tpu as pltpu
from jax.experimental.pallas import tpu_sc as plsc

_N = 10000
_D = 128
_NEI = 2
_S = 5
_NP = 10240
_QB = 512


def _flash_body(q_ref, k_ref, v_ref, ow_ref, ob_ref, o_ref):
    q = q_ref[0]
    k = k_ref[0]
    v = v_ref[0]
    s = jax.lax.dot_general(q, k, (((1,), (1,)), ((), ())),
                            preferred_element_type=jnp.float32)
    s = s * (1.0 / np.sqrt(_D).astype(np.float32))
    kidx = jax.lax.broadcasted_iota(jnp.int32, s.shape, 1)
    s = jnp.where(kidx < _N, s, -1e30)
    m = jnp.max(s, axis=-1, keepdims=True)
    p = jnp.exp(s - m)
    l = jnp.sum(p, axis=-1, keepdims=True)
    o = jax.lax.dot_general(p.astype(jnp.bfloat16), v, (((1,), (0,)), ((), ())),
                            preferred_element_type=jnp.float32) / l
    o = jax.lax.dot_general(o, ow_ref[0], (((1,), (1,)), ((), ())),
                            preferred_element_type=jnp.float32)
    o_ref[0] = o + ob_ref[0]


def _flash_attention(qkv, out_w, out_b):
    grid = (_NEI, _NP // _QB)
    return pl.pallas_call(
        _flash_body,
        grid=grid,
        in_specs=[
            pl.BlockSpec((1, _QB, _D), lambda i, qb: (i, qb, 0)),
            pl.BlockSpec((1, _NP, _D), lambda i, qb: (i, 0, 1)),
            pl.BlockSpec((1, _NP, _D), lambda i, qb: (i, 0, 2)),
            pl.BlockSpec((1, _D, _D), lambda i, qb: (i, 0, 0)),
            pl.BlockSpec((1, 1, _D), lambda i, qb: (i, 0, 0)),
        ],
        out_specs=pl.BlockSpec((1, _QB, _D), lambda i, qb: (i, qb, 0)),
        out_shape=jax.ShapeDtypeStruct((_NEI, _NP, _D), jnp.float32),
    )(qkv, qkv, qkv, out_w, out_b.reshape(_NEI, 1, _D))


_E = 160000
_TILES = 16
_CB = 80
_EPT = _E // _TILES
_NCH = _EPT // _CB
_NPAD = 10240
_RPT = _NPAD // _TILES
_ZR = 40
_NSEG = 5
_SCH = _NCH // _NSEG

_SC_MESH = plsc.VectorSubcoreMesh(core_axis_name="c", subcore_axis_name="s")


def _lane_splat(vec16, j):
    return lax.gather(
        vec16, jnp.full((16, 1), j, jnp.int32),
        lax.GatherDimensionNumbers(offset_dims=(), collapsed_slice_dims=(0,),
                                   start_index_map=(0,)),
        (1,), mode=lax.GatherScatterMode.PROMISE_IN_BOUNDS)


_NSP = 4
_INTRA_MESH = plsc.VectorSubcoreMesh(core_axis_name="c", subcore_axis_name="s")
_NPT = _NPAD // _TILES
_GC = 64


@functools.partial(
    pl.kernel,
    mesh=_INTRA_MESH,
    out_type=jax.ShapeDtypeStruct((_NEI, _NPAD, _D), jnp.float32),
    scratch_types=[
        pltpu.VMEM((_NPT * _S,), jnp.int32),
        pltpu.VMEM((_NPT * _S,), jnp.float32),
        pltpu.VMEM((_GC * _S, _D), jnp.float32),
        pltpu.VMEM((_GC, _D), jnp.float32),
        pltpu.VMEM((_GC, _D), jnp.float32),
        pltpu.SemaphoreType.DMA,
    ],
)
def _intra_sc(h_hbm, h0_hbm, nei_hbm, w_hbm, out_hbm,
              idx_slab, w_slab, rows_v, h0_v, out_v, sem):
    c = lax.axis_index("c")
    s = lax.axis_index("s")
    n0 = s * _NPT
    pltpu.sync_copy(nei_hbm.at[c, 0, pl.ds(n0 * _S, _NPT * _S)], idx_slab)
    pltpu.sync_copy(w_hbm.at[c, 0, pl.ds(n0 * _S, _NPT * _S)], w_slab)

    def chunk(t, carry):
        pltpu.async_copy(h_hbm.at[idx_slab.at[pl.ds(t * (_GC * _S),
                                                    _GC * _S)]],
                         rows_v, sem).wait()
        pltpu.sync_copy(h0_hbm.at[pl.ds(n0 + t * _GC, _GC)], h0_v)

        def groupC(g, carry2):
            wvs = [w_slab[pl.ds(t * (_GC * _S) + g * 80 + s5 * 16, 16)]
                   for s5 in range(_S)]
            for lj in range(16):
                ws = [_lane_splat(wvs[s5], lj) for s5 in range(_S)]
                r = g * 16 + lj
                for db in range(_D // 16):
                    acc = h0_v[r, pl.ds(db * 16, 16)]
                    for s5 in range(_S):
                        acc = acc + ws[s5] * rows_v[g * 80 + s5 * 16 + lj,
                                                    pl.ds(db * 16, 16)]
                    out_v[r, pl.ds(db * 16, 16)] = acc
            return carry2
        lax.fori_loop(0, _GC // 16, groupC, 0)
        pltpu.sync_copy(out_v, out_hbm.at[c, pl.ds(n0 + t * _GC, _GC)])
        return carry
    lax.fori_loop(0, _NPT // _GC, chunk, 0)


def _tile_order(arr):
    return arr.reshape(_NEI, _TILES, _NPT // _GC, _GC // 16, 16, _S
                       ).transpose(0, 1, 2, 3, 5, 4).reshape(
                           _NEI, 1, _NPAD * _S)


def _intra_pair(nei_h, nei_index, intra_att_w):
    h0 = nei_h[0]
    h2 = nei_h[1:].reshape(_NEI * _N, _D)
    a0 = jnp.einsum('nd,id->in', h0, intra_att_w[:, 0, :_D])
    hb = jnp.einsum('ind,id->in', nei_h[1:], intra_att_w[:, 0, _D:])
    hbg = jnp.stack([jnp.take(hb[i], nei_index[i]) for i in range(_NEI)])
    att = jax.nn.leaky_relu(a0[:, :, None] + hbg, negative_slope=0.01)
    w = jax.nn.softmax(att, axis=2)
    padn3 = ((0, 0), (0, _NPAD - _N), (0, 0))
    off = (jnp.arange(_NEI, dtype=jnp.int32) * _N)[:, None, None]
    idx = _tile_order(jnp.pad(nei_index + off, padn3))
    wso = _tile_order(jnp.pad(w, padn3))
    h0p = jnp.pad(h0, ((0, _NPAD - _N), (0, 0)))
    out = _intra_sc(h2, h0p, idx, wso)
    return out[:, :_N, :]


@functools.partial(
    pl.kernel,
    mesh=_SC_MESH,
    out_type=jax.ShapeDtypeStruct((_NEI, _NPAD, _D), jnp.float32),
    scratch_types=[
        pltpu.VMEM((_SCH, _CB), jnp.int32),
        pltpu.VMEM((_SCH, _CB), jnp.int32),
        pltpu.VMEM((_SCH, _CB), jnp.float32),
        pltpu.VMEM((_CB, _D), jnp.float32),
        pltpu.VMEM((_ZR, _D), jnp.float32),
        pltpu.VMEM_SHARED((_NPAD, _D), jnp.float32),
        pltpu.SemaphoreType.DMA,
    ],
)
def _spmm2_sc(x_hbm, src_hbm, dst_hbm, w_hbm, out_hbm,
              src_v, dst_v, w_v, rows_v, z_v, acc_sh, sem):
    c = lax.axis_index("c")
    s = lax.axis_index("s")

    def zrow(r, carry):
        for db in range(_D // 16):
            z_v[r, pl.ds(db * 16, 16)] = jnp.zeros((16,), jnp.float32)
        return carry
    lax.fori_loop(0, _ZR, zrow, 0)
    r0 = s * _RPT

    for kz in range(_RPT // _ZR):
        pltpu.sync_copy(z_v, acc_sh.at[pl.ds(r0 + kz * _ZR, _ZR)])
    plsc.subcore_barrier()

    def segment(g, carry0):
        pltpu.sync_copy(src_hbm.at[c, s, g], src_v)
        pltpu.sync_copy(dst_hbm.at[c, s, g], dst_v)
        pltpu.sync_copy(w_hbm.at[c, s, g], w_v)

        def chunk(k, carry):
            pltpu.async_copy(x_hbm.at[src_v.at[k]], rows_v, sem).wait()

            def sgroup(gg, carry2):
                wv = w_v[k, pl.ds(gg * 16, 16)]
                for j in range(16):
                    wbc = _lane_splat(wv, j)
                    r = gg * 16 + j
                    for db in range(_D // 16):
                        rows_v[r, pl.ds(db * 16, 16)] = (
                            rows_v[r, pl.ds(db * 16, 16)] * wbc)
                return carry2
            lax.fori_loop(0, _CB // 16, sgroup, 0)
            pltpu.sync_copy(rows_v, acc_sh.at[dst_v.at[k]], add=True)
            return carry
        lax.fori_loop(0, _SCH, chunk, 0)
        return carry0
    lax.fori_loop(0, _NSEG, segment, 0)

    plsc.subcore_barrier()
    pltpu.sync_copy(acc_sh.at[pl.ds(r0, _RPT)],
                    out_hbm.at[c, pl.ds(r0, _RPT)])


def _spmm_pair(x2, edges2, w2):
    off = jnp.arange(_NEI, dtype=jnp.int32)[:, None] * _N
    srcs = edges2[:, 0, :] + off
    dsts = edges2[:, 1, :]
    sh = (_NEI, _TILES, _NSEG, _SCH, _CB)
    return _spmm2_sc(x2.reshape(_NEI * _N, _D), srcs.reshape(sh),
                     dsts.reshape(sh), w2.reshape(sh))


_PB = 400


def _proj_body(af_ref, w_ref, o_ref):
    o_ref[0] = jax.lax.dot_general(
        af_ref[0], w_ref[0], (((1,), (0,)), ((), ())),
        preferred_element_type=jnp.float32)


def _proj2(af2, gcn_w, att_w):
    wcat = jnp.concatenate([att_w, gcn_w], axis=2)
    out = pl.pallas_call(
        _proj_body,
        grid=(_NEI, _N // _PB),
        in_specs=[
            pl.BlockSpec((1, _PB, _D), lambda i, nb: (i, nb, 0)),
            pl.BlockSpec((1, _D, 2 * _D), lambda i, nb: (i, 0, 0)),
        ],
        out_specs=pl.BlockSpec((1, _PB, 2 * _D), lambda i, nb: (i, nb, 0)),
        out_shape=jax.ShapeDtypeStruct((_NEI, _N, 2 * _D), jnp.float32),
    )(af2, wcat)
    return out[:, :, :_D], out[:, :, _D:]


def _qkv_body(x_ref, ab_ref, w_ref, b_ref, o_ref):
    x = x_ref[0] + ab_ref[0]
    o = jnp.where(x > 0, x, jnp.exp(x) - 1.0)
    o_ref[0] = (jax.lax.dot_general(
        o, w_ref[0], (((1,), (1,)), ((), ())),
        preferred_element_type=jnp.float32) + b_ref[0]).astype(jnp.bfloat16)


def _qkv(att_pre, att_b, in_w, in_b):
    return pl.pallas_call(
        _qkv_body,
        grid=(_NEI, _NP // _QB),
        in_specs=[
            pl.BlockSpec((1, _QB, _D), lambda i, nb: (i, nb, 0)),
            pl.BlockSpec((1, 1, _D), lambda i, nb: (i, 0, 0)),
            pl.BlockSpec((1, 3 * _D, _D), lambda i, nb: (i, 0, 0)),
            pl.BlockSpec((1, 1, 3 * _D), lambda i, nb: (i, 0, 0)),
        ],
        out_specs=pl.BlockSpec((1, _QB, 3 * _D), lambda i, nb: (i, nb, 0)),
        out_shape=jax.ShapeDtypeStruct((_NEI, _NP, 3 * _D), jnp.bfloat16),
    )(att_pre, att_b.reshape(_NEI, 1, _D), in_w,
      in_b.reshape(_NEI, 1, 3 * _D))


def _inter_body(gp_ref, dif_ref, gb_ref, fcw_ref, fcb_ref, attp_ref, o_ref,
                acc_ref, m_ref, l_ref):
    nb = pl.program_id(1)
    af = gp_ref[0] + gb_ref[0]
    dif = dif_ref[0]
    t1 = jnp.tanh(jax.lax.dot_general(
        af, fcw_ref[0], (((1,), (1,)), ((), ())),
        preferred_element_type=jnp.float32) + fcb_ref[0])
    t2 = jnp.tanh(jax.lax.dot_general(
        dif, fcw_ref[0], (((1,), (1,)), ((), ())),
        preferred_element_type=jnp.float32) + fcb_ref[0])
    sp = (t1 + t2) * 0.5
    z = jax.lax.dot_general(sp, attp_ref[0], (((1,), (1,)), ((), ())),
                            preferred_element_type=jnp.float32)

    @pl.when(nb == 0)
    def _():
        m_ref[0] = -1e30
        l_ref[0] = 0.0
        acc_ref[...] = jnp.zeros_like(acc_ref)

    m_old = m_ref[0]
    m_new = jnp.maximum(m_old, jnp.max(z))
    scale = jnp.exp(m_old - m_new)
    e = jnp.exp(z - m_new)
    m_ref[0] = m_new
    l_ref[0] = l_ref[0] * scale + jnp.sum(e)
    eaf = jax.lax.dot_general(e, af, (((0,), (0,)), ((), ())),
                              preferred_element_type=jnp.float32)
    edf = jax.lax.dot_general(e, dif, (((0,), (0,)), ((), ())),
                              preferred_element_type=jnp.float32)
    acc_ref[...] = acc_ref[...] * scale + jnp.concatenate([eaf, edf], axis=0)

    @pl.when(nb == pl.num_programs(1) - 1)
    def _():
        o_ref[0] = acc_ref[...] / l_ref[0]


def _inter2(gcn_pre, dif, gcn_b, fc_w, fc_b, attp):
    return pl.pallas_call(
        _inter_body,
        grid=(_NEI, _N // _PB),
        in_specs=[
            pl.BlockSpec((1, _PB, _D), lambda i, nb: (i, nb, 0)),
            pl.BlockSpec((1, _PB, _D), lambda i, nb: (i, nb, 0)),
            pl.BlockSpec((1, 1, _D), lambda i, nb: (i, 0, 0)),
            pl.BlockSpec((1, _D, _D), lambda i, nb: (i, 0, 0)),
            pl.BlockSpec((1, 1, _D), lambda i, nb: (i, 0, 0)),
            pl.BlockSpec((1, 1, _D), lambda i, nb: (i, 0, 0)),
        ],
        out_specs=pl.BlockSpec((1, 2, _D), lambda i, nb: (i, 0, 0)),
        out_shape=jax.ShapeDtypeStruct((_NEI, 2, _D), jnp.float32),
        scratch_shapes=[
            pltpu.VMEM((2, _D), jnp.float32),
            pltpu.SMEM((1,), jnp.float32),
            pltpu.SMEM((1,), jnp.float32),
        ],
    )(gcn_pre, dif, gcn_b.reshape(_NEI, 1, _D), fc_w,
      fc_b.reshape(_NEI, 1, _D), attp)


def _spmm_x(edges, w, x):
    return jnp.zeros((_N, x.shape[1]), x.dtype).at[edges[1]].add(
        w[:, None] * x[edges[0]])


def _intra_x(nei, h, h_ref, attw):
    nei_emb = jnp.take(h, nei, axis=0)
    hr = jnp.broadcast_to(h_ref[:, None, :], nei_emb.shape)
    all_emb = jnp.concatenate([hr, nei_emb], axis=-1)
    att = jax.nn.leaky_relu(all_emb @ attw.T, negative_slope=0.01)
    att = jax.nn.softmax(att, axis=1)
    return (att * nei_emb).sum(axis=1)


def _inter_x(embeds, fcw, fcb, attp):
    sp = jnp.tanh(embeds @ fcw.T + fcb).mean(axis=1)
    beta = jax.nn.softmax(sp @ attp[0], axis=0)
    return jnp.einsum('m,mkd->kd', beta, embeds)


def kernel(nei_h, nei_index, adj_edges, adj_trans_edges, adj_w, adj_trans_w,
           intra_att_w, inter_fc_w, inter_fc_b, inter_attp, gcn_w, gcn_b,
           att_w, att_b, mha_in_w, mha_in_b, mha_out_w, mha_out_b,
           final_fc_w, final_fc_b, final_attp):
    af2 = _intra_pair(nei_h, nei_index, intra_att_w)
    y2, x2 = _proj2(af2, gcn_w, att_w)
    att_pre = _spmm_pair(y2, adj_edges, adj_w)
    gcn_pre = _spmm_pair(x2, adj_trans_edges, adj_trans_w)

    qkv = _qkv(att_pre, att_b, mha_in_w, mha_in_b)
    dif = _flash_attention(qkv, mha_out_w, mha_out_b)

    multi = _inter2(gcn_pre, dif, gcn_b, inter_fc_w, inter_fc_b, inter_attp)
    final_in = multi.transpose(1, 0, 2)
    return _inter_x(final_in, final_fc_w, final_fc_b, final_attp)
